# baseline (speedup 1.0000x reference)
import functools

import jax
import jax.numpy as jnp
from jax import lax
from jax.experimental import pallas as pl
from jax.experimental.pallas import tpu as pltpu
from jax.experimental.pallas import tpu_sc as plsc

_NUM_EMB = 512

_BB = 8


def _vq_step(z_ref, emb2b_ref, z2_ref, e2_ref,
             idx_ref, loss_ref):
    emb2b = emb2b_ref[...]
    e2 = e2_ref[...]
    k = _NUM_EMB

    for i in range(_BB):
        zb = z_ref[i]
        z2 = z2_ref[i]
        hw = zb.shape[1]

        scores2 = lax.dot_general(
            emb2b, zb.astype(jnp.bfloat16), (((1,), (0,)), ((), ())),
            preferred_element_type=jnp.float32,
        )
        dist = (z2 + e2) - scores2

        minv = jnp.min(dist, axis=0, keepdims=True)
        kiota = lax.broadcasted_iota(jnp.int32, (k, hw), 0)
        idx = jnp.min(jnp.where(dist == minv, kiota, k),
                      axis=0, keepdims=True)

        idx_ref[i] = idx
        loss_ref[pl.program_id(0) * _BB + i, 0] = jnp.sum(minv)


def _sc_gather(table, idx):
    info = plsc.get_sparse_core_info()
    nw = info.num_cores * info.num_subcores
    n = idx.shape[0]
    d = table.shape[1]
    b_per_w = n // nw
    chunk = 256
    mesh = plsc.VectorSubcoreMesh(core_axis_name="c", subcore_axis_name="s")

    @functools.partial(
        pl.kernel, mesh=mesh,
        out_type=jax.ShapeDtypeStruct((n, d), jnp.float32),
        scratch_types=[
            pltpu.VMEM((chunk,), jnp.int32),
            pltpu.VMEM((chunk, d), jnp.float32),
            pltpu.SemaphoreType.DMA,
        ],
    )
    def k(table_hbm, idx_hbm, out_hbm, idx_v, rows_v, sem):
        wid = lax.axis_index("s") * info.num_cores + lax.axis_index("c")
        base = wid * b_per_w
        for j in range(b_per_w // chunk):
            off = base + j * chunk
            pltpu.sync_copy(idx_hbm.at[pl.ds(off, chunk)], idx_v)
            pltpu.async_copy(table_hbm.at[idx_v], rows_v, sem).wait()
            pltpu.sync_copy(rows_v, out_hbm.at[pl.ds(off, chunk)])

    return k(table, idx)


def kernel(z, embedding):
    B, C, H, W = z.shape
    HW = H * W
    K = embedding.shape[0]
    z3 = z.reshape(B, C, HW)
    z2 = jnp.sum(z * z, axis=1).reshape(B, 1, HW)
    e2 = jnp.sum(embedding * embedding, axis=1).reshape(K, 1)
    emb2b = (2.0 * embedding).astype(jnp.bfloat16)

    idx3, loss_parts = pl.pallas_call(
        _vq_step,
        grid=(B // _BB,),
        in_specs=[
            pl.BlockSpec((_BB, C, HW), lambda b: (b, 0, 0)),
            pl.BlockSpec((K, C), lambda b: (0, 0)),
            pl.BlockSpec((_BB, 1, HW), lambda b: (b, 0, 0)),
            pl.BlockSpec((K, 1), lambda b: (0, 0)),
        ],
        out_specs=[
            pl.BlockSpec((_BB, 1, HW), lambda b: (b, 0, 0)),
            pl.BlockSpec(memory_space=pltpu.SMEM, block_shape=(B, 1),
                         index_map=lambda b: (0, 0)),
        ],
        out_shape=[
            jax.ShapeDtypeStruct((B, 1, HW), jnp.int32),
            jax.ShapeDtypeStruct((B, 1), jnp.float32),
        ],
    )(z3, emb2b, z2, e2)

    idx_flat = idx3.reshape(B * HW)
    table_pad = jnp.pad(embedding, ((0, 0), (0, 128 - C)))
    qflat = _sc_gather(table_pad, idx_flat)
    quantized_st_t = (qflat[:, :C].reshape(B, HW, C)
                      .transpose(0, 2, 1).reshape(B, C, H, W))
    loss = jnp.sum(loss_parts) / (B * C * HW)
    encoding_indices = idx3.reshape(B, H, W)
    return quantized_st_t, loss, loss, encoding_indices


# speedup vs baseline: 1.3378x; 1.3378x over previous
import jax
import jax.numpy as jnp
from jax import lax
from jax.experimental import pallas as pl
from jax.experimental.pallas import tpu as pltpu

_NUM_EMB = 512

_BB = 8


def _vq_step(z_ref, emb2b_ref, embb_ref, z2_ref, e2_ref,
             q_ref, idx_ref, loss_ref):
    emb2b = emb2b_ref[...]
    embb = embb_ref[...]
    e2 = e2_ref[...]
    k = _NUM_EMB

    for i in range(_BB):
        zb = z_ref[i]
        z2 = z2_ref[i]
        hw = zb.shape[1]

        scores2 = lax.dot_general(
            emb2b, zb.astype(jnp.bfloat16), (((1,), (0,)), ((), ())),
            preferred_element_type=jnp.float32,
        )
        dist = (z2 + e2) - scores2

        minv = jnp.min(dist, axis=0, keepdims=True)
        idx = jnp.argmin(dist, axis=0).reshape(1, hw)
        kiota = lax.broadcasted_iota(jnp.int32, (k, hw), 0)

        onehot = (kiota == idx).astype(jnp.bfloat16)
        q = lax.dot_general(
            embb, onehot, (((0,), (0,)), ((), ())),
            preferred_element_type=jnp.float32,
        )

        q_ref[i] = q
        idx_ref[i] = idx
        loss_ref[pl.program_id(0) * _BB + i, 0] = jnp.sum(minv)


def kernel(z, embedding):
    B, C, H, W = z.shape
    HW = H * W
    K = embedding.shape[0]
    z3 = z.reshape(B, C, HW)
    z2 = jnp.sum(z * z, axis=1).reshape(B, 1, HW)
    e2 = jnp.sum(embedding * embedding, axis=1).reshape(K, 1)
    emb2b = (2.0 * embedding).astype(jnp.bfloat16)
    embb = embedding.astype(jnp.bfloat16)

    q3, idx3, loss_parts = pl.pallas_call(
        _vq_step,
        grid=(B // _BB,),
        in_specs=[
            pl.BlockSpec((_BB, C, HW), lambda b: (b, 0, 0)),
            pl.BlockSpec((K, C), lambda b: (0, 0)),
            pl.BlockSpec((K, C), lambda b: (0, 0)),
            pl.BlockSpec((_BB, 1, HW), lambda b: (b, 0, 0)),
            pl.BlockSpec((K, 1), lambda b: (0, 0)),
        ],
        out_specs=[
            pl.BlockSpec((_BB, C, HW), lambda b: (b, 0, 0)),
            pl.BlockSpec((_BB, 1, HW), lambda b: (b, 0, 0)),
            pl.BlockSpec(memory_space=pltpu.SMEM, block_shape=(B, 1),
                         index_map=lambda b: (0, 0)),
        ],
        out_shape=[
            jax.ShapeDtypeStruct((B, C, HW), jnp.float32),
            jax.ShapeDtypeStruct((B, 1, HW), jnp.int32),
            jax.ShapeDtypeStruct((B, 1), jnp.float32),
        ],
    )(z3, emb2b, embb, z2, e2)

    quantized_st_t = q3.reshape(B, C, H, W)
    loss = jnp.sum(loss_parts) / (B * C * HW)
    encoding_indices = idx3.reshape(B, H, W)
    return quantized_st_t, loss, loss, encoding_indices
